# baseline (speedup 1.0000x reference)
import functools

import jax
import jax.numpy as jnp
from jax import lax
from jax.experimental import pallas as pl
from jax.experimental.pallas import tpu as pltpu
from jax.experimental.pallas import tpu_sc as plsc

_N = 10000
_E = 320000
_DF = 128
_DE = 16
_DO = 128

_NC = 2
_NS = 16
_NW = _NC * _NS
_EPW = _E // _NW
_K = 80
_NCHUNK = _EPW // _K
_KB = 40
_RPS = 624
_TAIL = _N - _RPS * _NS
_TOFF = _RPS * _NS
_PIECES = tuple((80 * p, 80) for p in range(7)) + ((560, 64),)

_SELU_ALPHA = 1.6732632423543772
_SELU_SCALE = 1.0507009873554805


def _z16():
    return jnp.zeros((16,), jnp.float32)


def _sc_body(feat_hbm, src_hbm, dst_hbm, attr_hbm, out_f, out_a,
             ixs0, ixs1, ixs2, ixd0, ixd1, ixd2, r0, r1, r2, trows,
             ib0, ib1, ib2, a0, a1, a2, acc,
             isem0, isem1, isem2, gsem0, gsem1, gsem2,
             ssem0, ssem1, ssem2):
    c = lax.axis_index("c")
    s = lax.axis_index("s")
    rb = s * _RPS
    wbase = (c * _NS + s) * _EPW

    IXS = (ixs0, ixs1, ixs2)
    IXD = (ixd0, ixd1, ixd2)
    R = (r0, r1, r2)
    IXB = (ib0, ib1, ib2)
    A16 = (a0, a1, a2)
    ISEM = (isem0, isem1, isem2)
    GSEM = (gsem0, gsem1, gsem2)
    SSEM = (ssem0, ssem1, ssem2)

    def zero_r0():
        for i in range(_K):
            for k in range(_DF // 16):
                r0[i, pl.ds(16 * k, 16)] = _z16()

    def zero_acc():
        for lo, nr in _PIECES:
            pltpu.sync_copy(r0.at[pl.ds(0, nr)], acc.at[pl.ds(rb + lo, nr)])

        @pl.when(s == 0)
        def _tail():
            pltpu.sync_copy(r0.at[pl.ds(0, _TAIL)],
                            acc.at[pl.ds(_TOFF, _TAIL)])

    def drain_acc(out):
        for p, (lo, nr) in enumerate(_PIECES):
            buf, sem = R[p % 2], GSEM[p % 2]
            if p >= 2:
                plo, pnr = _PIECES[p - 2]
                pltpu.make_async_copy(
                    buf.at[pl.ds(0, pnr)],
                    out.at[c, pl.ds(rb + plo, pnr)], sem).wait()
            pltpu.sync_copy(acc.at[pl.ds(rb + lo, nr)], buf.at[pl.ds(0, nr)])
            pltpu.async_copy(buf.at[pl.ds(0, nr)],
                             out.at[c, pl.ds(rb + lo, nr)], sem)
        for p in (len(_PIECES) - 2, len(_PIECES) - 1):
            lo, nr = _PIECES[p]
            buf, sem = R[p % 2], GSEM[p % 2]
            pltpu.make_async_copy(buf.at[pl.ds(0, nr)],
                                  out.at[c, pl.ds(rb + lo, nr)], sem).wait()

        @pl.when(s == 0)
        def _tail():
            pltpu.sync_copy(acc.at[pl.ds(_TOFF, _TAIL)], trows)
            pltpu.sync_copy(trows, out.at[c, pl.ds(_TOFF, _TAIL)])

    def w_scat(u):
        pltpu.make_async_copy(R[u], acc.at[IXD[u]], SSEM[u]).wait()

    def i_scat(u):
        pltpu.async_copy(R[u], acc.at[IXD[u]], SSEM[u], add=True)

    def i_idx_a(u, off):
        pltpu.async_copy(src_hbm.at[pl.ds(off, _K)], IXS[u], ISEM[u])
        pltpu.async_copy(dst_hbm.at[pl.ds(off, _K)], IXD[u], ISEM[u])

    def w_idx_a(u, off):
        pltpu.make_async_copy(src_hbm.at[pl.ds(off, _K)], IXS[u],
                              ISEM[u]).wait()
        pltpu.make_async_copy(dst_hbm.at[pl.ds(off, _K)], IXD[u],
                              ISEM[u]).wait()

    def i_gath(u):
        pltpu.async_copy(feat_hbm.at[IXS[u]], R[u], GSEM[u])

    def w_gath(u):
        pltpu.make_async_copy(feat_hbm.at[IXS[u]], R[u], GSEM[u]).wait()

    def i_idx_b(u, off):
        pltpu.async_copy(dst_hbm.at[pl.ds(off, _KB)], IXB[u], ISEM[u])
        pltpu.async_copy(attr_hbm.at[pl.ds(off, _KB)], A16[u], GSEM[u])

    def w_idx_b(u, off):
        pltpu.make_async_copy(dst_hbm.at[pl.ds(off, _KB)], IXB[u],
                              ISEM[u]).wait()
        pltpu.make_async_copy(attr_hbm.at[pl.ds(off, _KB)], A16[u],
                              GSEM[u]).wait()

    def w_scat_b(u):
        pltpu.make_async_copy(R[u].at[pl.ds(0, _KB)], acc.at[IXB[u]],
                              SSEM[u]).wait()

    def i_scat_b(u):
        pltpu.async_copy(R[u].at[pl.ds(0, _KB)], acc.at[IXB[u]], SSEM[u],
                         add=True)

    zero_r0()
    zero_acc()
    plsc.subcore_barrier()

    def body_a(t, carry):
        for u in range(3):
            i_base = 3 * t + u

            cond1 = t >= 1
            if u == 2:
                cond1 = jnp.logical_and(cond1, t <= 41)

            @pl.when(cond1)
            def _s1(u=u):
                w_scat(u)

            cond2 = t <= (41 if u <= 1 else 40)

            @pl.when(cond2)
            def _s2(u=u, i_base=i_base):
                i_idx_a(u, wbase + i_base * _K)

            v = (u - 1) % 3
            cond3 = t <= 41
            if u == 0:
                cond3 = jnp.logical_and(cond3, t >= 1)

            @pl.when(cond3)
            def _s3(v=v, i_base=i_base):
                w_idx_a(v, wbase + (i_base - 1) * _K)
                i_gath(v)

            w = (u - 2) % 3
            if u == 0:
                cond4 = jnp.logical_and(t >= 1, t <= 42)
            elif u == 1:
                cond4 = jnp.logical_and(t >= 1, t <= 41)
            else:
                cond4 = t <= 41

            @pl.when(cond4)
            def _s4(w=w):
                w_gath(w)
                i_scat(w)
        return carry

    lax.fori_loop(0, 43, body_a, 0)
    plsc.subcore_barrier()
    drain_acc(out_f)

    zero_r0()
    zero_acc()
    def fillb(i, carry):
        for k in range(1, _DF // 16):
            one = jnp.full((16,), jnp.float32(1.0) if k == 1 else 0.0,
                           jnp.float32)
            r0[i, pl.ds(16 * k, 16)] = one
            r1[i, pl.ds(16 * k, 16)] = one
            r2[i, pl.ds(16 * k, 16)] = one
        return carry

    lax.fori_loop(0, _KB, fillb, 0)
    plsc.subcore_barrier()

    def body_b(t, carry):
        for u in range(3):
            i_base = 3 * t + u

            if u == 0:
                cond1 = jnp.logical_and(t >= 1, t <= 84)
            else:
                cond1 = jnp.logical_and(t >= 1, t <= 83)

            @pl.when(cond1)
            def _s1(u=u):
                w_scat_b(u)

            cond2 = t <= (83 if u == 0 else 82)

            @pl.when(cond2)
            def _s2(u=u, i_base=i_base):
                i_idx_b(u, wbase + i_base * _KB)

            v = (u - 1) % 3
            if u == 0:
                cond3 = jnp.logical_and(t >= 1, t <= 83)
            elif u == 1:
                cond3 = t <= 83
            else:
                cond3 = t <= 82

            @pl.when(cond3)
            def _s3(u=u, v=v, i_base=i_base):
                w_idx_b(v, wbase + (i_base - 1) * _KB)
                for i in range(_KB):
                    R[v][i, pl.ds(0, _DE)] = A16[v][i, :]
                i_scat_b(v)
        return carry

    lax.fori_loop(0, 85, body_b, 0)
    plsc.subcore_barrier()
    drain_acc(out_a)


_sc_call = functools.partial(
    pl.kernel,
    out_type=[
        jax.ShapeDtypeStruct((_NC, _N, _DF), jnp.float32),
        jax.ShapeDtypeStruct((_NC, _N, _DF), jnp.float32),
    ],
    mesh=plsc.VectorSubcoreMesh(core_axis_name="c", subcore_axis_name="s"),
    scratch_types=[
        pltpu.VMEM((_K,), jnp.int32),
        pltpu.VMEM((_K,), jnp.int32),
        pltpu.VMEM((_K,), jnp.int32),
        pltpu.VMEM((_K,), jnp.int32),
        pltpu.VMEM((_K,), jnp.int32),
        pltpu.VMEM((_K,), jnp.int32),
        pltpu.VMEM((_K, _DF), jnp.float32),
        pltpu.VMEM((_K, _DF), jnp.float32),
        pltpu.VMEM((_K, _DF), jnp.float32),
        pltpu.VMEM((_TAIL, _DF), jnp.float32),
        pltpu.VMEM((_KB,), jnp.int32),
        pltpu.VMEM((_KB,), jnp.int32),
        pltpu.VMEM((_KB,), jnp.int32),
        pltpu.VMEM((_KB, _DE), jnp.float32),
        pltpu.VMEM((_KB, _DE), jnp.float32),
        pltpu.VMEM((_KB, _DE), jnp.float32),
        pltpu.VMEM_SHARED((_N, _DF), jnp.float32),
        pltpu.SemaphoreType.DMA,
        pltpu.SemaphoreType.DMA,
        pltpu.SemaphoreType.DMA,
        pltpu.SemaphoreType.DMA,
        pltpu.SemaphoreType.DMA,
        pltpu.SemaphoreType.DMA,
        pltpu.SemaphoreType.DMA,
        pltpu.SemaphoreType.DMA,
        pltpu.SemaphoreType.DMA,
    ],
)(_sc_body)


_ROWS = 1000


def _tc_body(pf, pa, w, b, out):
    sf = pf[0] + pf[1]
    sa = pa[0] + pa[1]
    se = sa[:, 0:_DE]
    deg = sa[:, _DE:_DE + 1]
    inv = 1.0 / jnp.maximum(deg, 1.0)
    h = jnp.dot(sf, w[_DE:, :], preferred_element_type=jnp.float32)
    h = h + jnp.dot(se, w[:_DE, :], preferred_element_type=jnp.float32)
    h = h * inv + b[...]
    neg = _SELU_ALPHA * (jnp.exp(jnp.minimum(h, 0.0)) - 1.0)
    out[...] = _SELU_SCALE * jnp.where(h > 0.0, h, neg)


_tc_call = pl.pallas_call(
    _tc_body,
    grid=(_N // _ROWS,),
    in_specs=[
        pl.BlockSpec((_NC, _ROWS, _DF), lambda i: (0, i, 0)),
        pl.BlockSpec((_NC, _ROWS, _DF), lambda i: (0, i, 0)),
        pl.BlockSpec((_DE + _DF, _DO), lambda i: (0, 0)),
        pl.BlockSpec((1, _DO), lambda i: (0, 0)),
    ],
    out_specs=pl.BlockSpec((_ROWS, _DO), lambda i: (i, 0)),
    out_shape=jax.ShapeDtypeStruct((_N, _DO), jnp.float32),
)


def kernel(feat, edge_index, edge_attr, weight, bias):
    src = edge_index[0]
    dst = edge_index[1]
    pf, pa = _sc_call(feat, src, dst, edge_attr)
    return _tc_call(pf, pa, weight, bias.reshape(1, _DO))


# speedup vs baseline: 1.0061x; 1.0061x over previous
import functools

import jax
import jax.numpy as jnp
from jax import lax
from jax.experimental import pallas as pl
from jax.experimental.pallas import tpu as pltpu
from jax.experimental.pallas import tpu_sc as plsc

_N = 10000
_E = 320000
_DF = 128
_DE = 16
_DO = 128

_NC = 2
_NS = 16
_NW = _NC * _NS
_EPW = _E // _NW
_K = 80
_NCHUNK = _EPW // _K
_RPS = 624
_TAIL = _N - _RPS * _NS
_TOFF = _RPS * _NS
_PIECES = tuple((80 * p, 80) for p in range(7)) + ((560, 64),)

_SELU_ALPHA = 1.6732632423543772
_SELU_SCALE = 1.0507009873554805


def _z16():
    return jnp.zeros((16,), jnp.float32)


def _sc_body(feat_hbm, src_hbm, dst_hbm, attr128_hbm, out_f, out_a,
             ixs0, ixs1, ixs2, ixd0, ixd1, ixd2, r0, r1, r2, trows, acc,
             isem0, isem1, isem2, gsem0, gsem1, gsem2,
             ssem0, ssem1, ssem2):
    c = lax.axis_index("c")
    s = lax.axis_index("s")
    rb = s * _RPS
    wbase = (c * _NS + s) * _EPW

    IXS = (ixs0, ixs1, ixs2)
    IXD = (ixd0, ixd1, ixd2)
    R = (r0, r1, r2)
    ISEM = (isem0, isem1, isem2)
    GSEM = (gsem0, gsem1, gsem2)
    SSEM = (ssem0, ssem1, ssem2)

    def zero_r0():
        for i in range(_K):
            for k in range(_DF // 16):
                r0[i, pl.ds(16 * k, 16)] = _z16()

    def zero_acc():
        for lo, nr in _PIECES:
            pltpu.sync_copy(r0.at[pl.ds(0, nr)], acc.at[pl.ds(rb + lo, nr)])

        @pl.when(s == 0)
        def _tail():
            pltpu.sync_copy(r0.at[pl.ds(0, _TAIL)],
                            acc.at[pl.ds(_TOFF, _TAIL)])

    def drain_acc(out):
        for p, (lo, nr) in enumerate(_PIECES):
            buf, sem = R[p % 2], GSEM[p % 2]
            if p >= 2:
                plo, pnr = _PIECES[p - 2]
                pltpu.make_async_copy(
                    buf.at[pl.ds(0, pnr)],
                    out.at[c, pl.ds(rb + plo, pnr)], sem).wait()
            pltpu.sync_copy(acc.at[pl.ds(rb + lo, nr)], buf.at[pl.ds(0, nr)])
            pltpu.async_copy(buf.at[pl.ds(0, nr)],
                             out.at[c, pl.ds(rb + lo, nr)], sem)
        for p in (len(_PIECES) - 2, len(_PIECES) - 1):
            lo, nr = _PIECES[p]
            buf, sem = R[p % 2], GSEM[p % 2]
            pltpu.make_async_copy(buf.at[pl.ds(0, nr)],
                                  out.at[c, pl.ds(rb + lo, nr)], sem).wait()

        @pl.when(s == 0)
        def _tail():
            pltpu.sync_copy(acc.at[pl.ds(_TOFF, _TAIL)], trows)
            pltpu.sync_copy(trows, out.at[c, pl.ds(_TOFF, _TAIL)])

    def w_scat(u):
        pltpu.make_async_copy(R[u], acc.at[IXD[u]], SSEM[u]).wait()

    def i_scat(u):
        pltpu.async_copy(R[u], acc.at[IXD[u]], SSEM[u], add=True)

    def i_idx_a(u, off):
        pltpu.async_copy(src_hbm.at[pl.ds(off, _K)], IXS[u], ISEM[u])
        pltpu.async_copy(dst_hbm.at[pl.ds(off, _K)], IXD[u], ISEM[u])

    def w_idx_a(u, off):
        pltpu.make_async_copy(src_hbm.at[pl.ds(off, _K)], IXS[u],
                              ISEM[u]).wait()
        pltpu.make_async_copy(dst_hbm.at[pl.ds(off, _K)], IXD[u],
                              ISEM[u]).wait()

    def i_gath(u):
        pltpu.async_copy(feat_hbm.at[IXS[u]], R[u], GSEM[u])

    def w_gath(u):
        pltpu.make_async_copy(feat_hbm.at[IXS[u]], R[u], GSEM[u]).wait()

    def i_idx_b(u, off):
        pltpu.async_copy(dst_hbm.at[pl.ds(off, _K)], IXD[u], ISEM[u])
        pltpu.async_copy(attr128_hbm.at[pl.ds(off, _K)], R[u], GSEM[u])

    def w_idx_b(u, off):
        pltpu.make_async_copy(dst_hbm.at[pl.ds(off, _K)], IXD[u],
                              ISEM[u]).wait()
        pltpu.make_async_copy(attr128_hbm.at[pl.ds(off, _K)], R[u],
                              GSEM[u]).wait()

    zero_r0()
    zero_acc()
    plsc.subcore_barrier()

    def body_a(t, carry):
        for u in range(3):
            i_base = 3 * t + u

            cond1 = t >= 1
            if u == 2:
                cond1 = jnp.logical_and(cond1, t <= 41)

            @pl.when(cond1)
            def _s1(u=u):
                w_scat(u)

            cond2 = t <= (41 if u <= 1 else 40)

            @pl.when(cond2)
            def _s2(u=u, i_base=i_base):
                i_idx_a(u, wbase + i_base * _K)

            v = (u - 1) % 3
            cond3 = t <= 41
            if u == 0:
                cond3 = jnp.logical_and(cond3, t >= 1)

            @pl.when(cond3)
            def _s3(v=v, i_base=i_base):
                w_idx_a(v, wbase + (i_base - 1) * _K)
                i_gath(v)

            w = (u - 2) % 3
            if u == 0:
                cond4 = jnp.logical_and(t >= 1, t <= 42)
            elif u == 1:
                cond4 = jnp.logical_and(t >= 1, t <= 41)
            else:
                cond4 = t <= 41

            @pl.when(cond4)
            def _s4(w=w):
                w_gath(w)
                i_scat(w)
        return carry

    lax.fori_loop(0, 43, body_a, 0)
    plsc.subcore_barrier()
    drain_acc(out_f)

    zero_r0()
    zero_acc()
    plsc.subcore_barrier()

    def body_b(t, carry):
        for u in range(3):
            i_base = 3 * t + u

            cond1 = t >= 1
            if u == 2:
                cond1 = jnp.logical_and(cond1, t <= 41)

            @pl.when(cond1)
            def _s1(u=u):
                w_scat(u)

            cond2 = t <= (41 if u <= 1 else 40)

            @pl.when(cond2)
            def _s2(u=u, i_base=i_base):
                i_idx_b(u, wbase + i_base * _K)

            v = (u - 1) % 3
            cond3 = t <= 41
            if u == 0:
                cond3 = jnp.logical_and(cond3, t >= 1)

            @pl.when(cond3)
            def _s3(v=v, i_base=i_base):
                w_idx_b(v, wbase + (i_base - 1) * _K)
                i_scat(v)
        return carry

    lax.fori_loop(0, 43, body_b, 0)
    plsc.subcore_barrier()
    drain_acc(out_a)


_sc_call = functools.partial(
    pl.kernel,
    out_type=[
        jax.ShapeDtypeStruct((_NC, _N, _DF), jnp.float32),
        jax.ShapeDtypeStruct((_NC, _N, _DF), jnp.float32),
    ],
    mesh=plsc.VectorSubcoreMesh(core_axis_name="c", subcore_axis_name="s"),
    scratch_types=[
        pltpu.VMEM((_K,), jnp.int32),
        pltpu.VMEM((_K,), jnp.int32),
        pltpu.VMEM((_K,), jnp.int32),
        pltpu.VMEM((_K,), jnp.int32),
        pltpu.VMEM((_K,), jnp.int32),
        pltpu.VMEM((_K,), jnp.int32),
        pltpu.VMEM((_K, _DF), jnp.float32),
        pltpu.VMEM((_K, _DF), jnp.float32),
        pltpu.VMEM((_K, _DF), jnp.float32),
        pltpu.VMEM((_TAIL, _DF), jnp.float32),
        pltpu.VMEM_SHARED((_N, _DF), jnp.float32),
        pltpu.SemaphoreType.DMA,
        pltpu.SemaphoreType.DMA,
        pltpu.SemaphoreType.DMA,
        pltpu.SemaphoreType.DMA,
        pltpu.SemaphoreType.DMA,
        pltpu.SemaphoreType.DMA,
        pltpu.SemaphoreType.DMA,
        pltpu.SemaphoreType.DMA,
        pltpu.SemaphoreType.DMA,
    ],
)(_sc_body)


_ROWS = 1000


def _tc_body(pf, pa, w, b, out):
    sf = pf[0] + pf[1]
    sa = pa[0] + pa[1]
    se = sa[:, 0:_DE]
    deg = sa[:, _DE:_DE + 1]
    inv = 1.0 / jnp.maximum(deg, 1.0)
    h = jnp.dot(sf, w[_DE:, :], preferred_element_type=jnp.float32)
    h = h + jnp.dot(se, w[:_DE, :], preferred_element_type=jnp.float32)
    h = h * inv + b[...]
    neg = _SELU_ALPHA * (jnp.exp(jnp.minimum(h, 0.0)) - 1.0)
    out[...] = _SELU_SCALE * jnp.where(h > 0.0, h, neg)


_tc_call = pl.pallas_call(
    _tc_body,
    grid=(_N // _ROWS,),
    in_specs=[
        pl.BlockSpec((_NC, _ROWS, _DF), lambda i: (0, i, 0)),
        pl.BlockSpec((_NC, _ROWS, _DF), lambda i: (0, i, 0)),
        pl.BlockSpec((_DE + _DF, _DO), lambda i: (0, 0)),
        pl.BlockSpec((1, _DO), lambda i: (0, 0)),
    ],
    out_specs=pl.BlockSpec((_ROWS, _DO), lambda i: (i, 0)),
    out_shape=jax.ShapeDtypeStruct((_N, _DO), jnp.float32),
)


def kernel(feat, edge_index, edge_attr, weight, bias):
    src = edge_index[0]
    dst = edge_index[1]
    attr128 = jnp.concatenate(
        [edge_attr,
         jnp.ones((_E, _DE), jnp.float32),
         jnp.zeros((_E, _DF - 2 * _DE), jnp.float32)], axis=1)
    pf, pa = _sc_call(feat, src, dst, attr128)
    return _tc_call(pf, pa, weight, bias.reshape(1, _DO))


# speedup vs baseline: 1.0122x; 1.0061x over previous
import functools

import jax
import jax.numpy as jnp
from jax import lax
from jax.experimental import pallas as pl
from jax.experimental.pallas import tpu as pltpu
from jax.experimental.pallas import tpu_sc as plsc

_N = 10000
_E = 320000
_DF = 128
_DE = 16
_DO = 128

_NC = 2
_NS = 16
_NW = _NC * _NS
_EPW = _E // _NW
_K = 80
_NCHUNK = _EPW // _K
_RPS = 624
_TAIL = _N - _RPS * _NS
_TOFF = _RPS * _NS
_PIECES = tuple((80 * p, 80) for p in range(7)) + ((560, 64),)

_SELU_ALPHA = 1.6732632423543772
_SELU_SCALE = 1.0507009873554805


def _z16():
    return jnp.zeros((16,), jnp.float32)


def _sc_body(feat_hbm, src_hbm, dst_hbm, attr128_hbm, out_f, out_a,
             ixs0, ixs1, ixs2, ixs3, ixd0, ixd1, ixd2, ixd3,
             r0, r1, r2, r3, trows, acc,
             isem0, isem1, isem2, isem3, gsem0, gsem1, gsem2, gsem3,
             ssem0, ssem1, ssem2, ssem3):
    c = lax.axis_index("c")
    s = lax.axis_index("s")
    rb = s * _RPS
    wbase = (c * _NS + s) * _EPW

    IXS = (ixs0, ixs1, ixs2, ixs3)
    IXD = (ixd0, ixd1, ixd2, ixd3)
    R = (r0, r1, r2, r3)
    ISEM = (isem0, isem1, isem2, isem3)
    GSEM = (gsem0, gsem1, gsem2, gsem3)
    SSEM = (ssem0, ssem1, ssem2, ssem3)

    def zero_r0():
        for i in range(_K):
            for k in range(_DF // 16):
                r0[i, pl.ds(16 * k, 16)] = _z16()

    def zero_acc():
        for lo, nr in _PIECES:
            pltpu.sync_copy(r0.at[pl.ds(0, nr)], acc.at[pl.ds(rb + lo, nr)])

        @pl.when(s == 0)
        def _tail():
            pltpu.sync_copy(r0.at[pl.ds(0, _TAIL)],
                            acc.at[pl.ds(_TOFF, _TAIL)])

    def drain_acc(out):
        for p, (lo, nr) in enumerate(_PIECES):
            buf, sem = R[p % 2], GSEM[p % 2]
            if p >= 2:
                plo, pnr = _PIECES[p - 2]
                pltpu.make_async_copy(
                    buf.at[pl.ds(0, pnr)],
                    out.at[c, pl.ds(rb + plo, pnr)], sem).wait()
            pltpu.sync_copy(acc.at[pl.ds(rb + lo, nr)], buf.at[pl.ds(0, nr)])
            pltpu.async_copy(buf.at[pl.ds(0, nr)],
                             out.at[c, pl.ds(rb + lo, nr)], sem)
        for p in (len(_PIECES) - 2, len(_PIECES) - 1):
            lo, nr = _PIECES[p]
            buf, sem = R[p % 2], GSEM[p % 2]
            pltpu.make_async_copy(buf.at[pl.ds(0, nr)],
                                  out.at[c, pl.ds(rb + lo, nr)], sem).wait()

        @pl.when(s == 0)
        def _tail():
            pltpu.sync_copy(acc.at[pl.ds(_TOFF, _TAIL)], trows)
            pltpu.sync_copy(trows, out.at[c, pl.ds(_TOFF, _TAIL)])

    def w_scat(u):
        pltpu.make_async_copy(R[u], acc.at[IXD[u]], SSEM[u]).wait()

    def i_scat(u):
        pltpu.async_copy(R[u], acc.at[IXD[u]], SSEM[u], add=True)

    def i_idx_a(u, off):
        pltpu.async_copy(src_hbm.at[pl.ds(off, _K)], IXS[u], ISEM[u])
        pltpu.async_copy(dst_hbm.at[pl.ds(off, _K)], IXD[u], ISEM[u])

    def w_idx_a(u, off):
        pltpu.make_async_copy(src_hbm.at[pl.ds(off, _K)], IXS[u],
                              ISEM[u]).wait()
        pltpu.make_async_copy(dst_hbm.at[pl.ds(off, _K)], IXD[u],
                              ISEM[u]).wait()

    def i_gath(u):
        pltpu.async_copy(feat_hbm.at[IXS[u]], R[u], GSEM[u])

    def w_gath(u):
        pltpu.make_async_copy(feat_hbm.at[IXS[u]], R[u], GSEM[u]).wait()

    def i_idx_b(u, off):
        pltpu.async_copy(dst_hbm.at[pl.ds(off, _K)], IXD[u], ISEM[u])
        pltpu.async_copy(attr128_hbm.at[pl.ds(off, _K)], R[u], GSEM[u])

    def w_idx_b(u, off):
        pltpu.make_async_copy(dst_hbm.at[pl.ds(off, _K)], IXD[u],
                              ISEM[u]).wait()
        pltpu.make_async_copy(attr128_hbm.at[pl.ds(off, _K)], R[u],
                              GSEM[u]).wait()

    zero_r0()
    zero_acc()
    plsc.subcore_barrier()

    def body_a(t, carry):
        for u in range(4):
            i_base = 4 * t + u

            cond1 = jnp.logical_and(t >= 1, t <= (32 if u == 0 else 31))

            @pl.when(cond1)
            def _s1(u=u):
                w_scat(u)

            cond2 = t <= (31 if u == 0 else 30)

            @pl.when(cond2)
            def _s2(u=u, i_base=i_base):
                i_idx_a(u, wbase + i_base * _K)

            v = (u - 1) % 4
            if u == 0:
                cond3 = jnp.logical_and(t >= 1, t <= 31)
            elif u == 1:
                cond3 = t <= 31
            else:
                cond3 = t <= 30

            @pl.when(cond3)
            def _s3(v=v, i_base=i_base):
                w_idx_a(v, wbase + (i_base - 1) * _K)
                i_gath(v)

            w = (u - 2) % 4
            if u == 0 or u == 1:
                cond4 = jnp.logical_and(t >= 1, t <= 31)
            elif u == 2:
                cond4 = t <= 31
            else:
                cond4 = t <= 30

            @pl.when(cond4)
            def _s4(w=w):
                w_gath(w)
                i_scat(w)
        return carry

    lax.fori_loop(0, 33, body_a, 0)
    plsc.subcore_barrier()
    drain_acc(out_f)

    zero_r0()
    zero_acc()
    plsc.subcore_barrier()

    def body_b(t, carry):
        for u in range(4):
            i_base = 4 * t + u

            cond1 = jnp.logical_and(t >= 1, t <= (32 if u == 0 else 31))

            @pl.when(cond1)
            def _s1(u=u):
                w_scat(u)

            cond2 = t <= (31 if u == 0 else 30)

            @pl.when(cond2)
            def _s2(u=u, i_base=i_base):
                i_idx_b(u, wbase + i_base * _K)

            v = (u - 1) % 4
            if u == 0:
                cond3 = jnp.logical_and(t >= 1, t <= 31)
            elif u == 1:
                cond3 = t <= 31
            else:
                cond3 = t <= 30

            @pl.when(cond3)
            def _s3(v=v, i_base=i_base):
                w_idx_b(v, wbase + (i_base - 1) * _K)
                i_scat(v)
        return carry

    lax.fori_loop(0, 33, body_b, 0)
    plsc.subcore_barrier()
    drain_acc(out_a)


_sc_call = functools.partial(
    pl.kernel,
    out_type=[
        jax.ShapeDtypeStruct((_NC, _N, _DF), jnp.float32),
        jax.ShapeDtypeStruct((_NC, _N, _DF), jnp.float32),
    ],
    mesh=plsc.VectorSubcoreMesh(core_axis_name="c", subcore_axis_name="s"),
    scratch_types=[
        pltpu.VMEM((_K,), jnp.int32),
        pltpu.VMEM((_K,), jnp.int32),
        pltpu.VMEM((_K,), jnp.int32),
        pltpu.VMEM((_K,), jnp.int32),
        pltpu.VMEM((_K,), jnp.int32),
        pltpu.VMEM((_K,), jnp.int32),
        pltpu.VMEM((_K,), jnp.int32),
        pltpu.VMEM((_K,), jnp.int32),
        pltpu.VMEM((_K, _DF), jnp.float32),
        pltpu.VMEM((_K, _DF), jnp.float32),
        pltpu.VMEM((_K, _DF), jnp.float32),
        pltpu.VMEM((_K, _DF), jnp.float32),
        pltpu.VMEM((_TAIL, _DF), jnp.float32),
        pltpu.VMEM_SHARED((_N, _DF), jnp.float32),
    ] + [pltpu.SemaphoreType.DMA] * 12,
)(_sc_body)


_ROWS = 1000


def _tc_body(pf, pa, w, b, out):
    sf = pf[0] + pf[1]
    sa = pa[0] + pa[1]
    se = sa[:, 0:_DE]
    deg = sa[:, _DE:_DE + 1]
    inv = 1.0 / jnp.maximum(deg, 1.0)
    h = jnp.dot(sf, w[_DE:, :], preferred_element_type=jnp.float32)
    h = h + jnp.dot(se, w[:_DE, :], preferred_element_type=jnp.float32)
    h = h * inv + b[...]
    neg = _SELU_ALPHA * (jnp.exp(jnp.minimum(h, 0.0)) - 1.0)
    out[...] = _SELU_SCALE * jnp.where(h > 0.0, h, neg)


_tc_call = pl.pallas_call(
    _tc_body,
    grid=(_N // _ROWS,),
    in_specs=[
        pl.BlockSpec((_NC, _ROWS, _DF), lambda i: (0, i, 0)),
        pl.BlockSpec((_NC, _ROWS, _DF), lambda i: (0, i, 0)),
        pl.BlockSpec((_DE + _DF, _DO), lambda i: (0, 0)),
        pl.BlockSpec((1, _DO), lambda i: (0, 0)),
    ],
    out_specs=pl.BlockSpec((_ROWS, _DO), lambda i: (i, 0)),
    out_shape=jax.ShapeDtypeStruct((_N, _DO), jnp.float32),
)


def kernel(feat, edge_index, edge_attr, weight, bias):
    src = edge_index[0]
    dst = edge_index[1]
    attr128 = jnp.concatenate(
        [edge_attr,
         jnp.ones((_E, _DE), jnp.float32),
         jnp.zeros((_E, _DF - 2 * _DE), jnp.float32)], axis=1)
    pf, pa = _sc_call(feat, src, dst, attr128)
    return _tc_call(pf, pa, weight, bias.reshape(1, _DO))
